# Initial kernel scaffold; baseline (speedup 1.0000x reference)
import jax
import jax.numpy as jnp
from jax.experimental import pallas as pl


def kernel(x, edge_index, batch, params):
    raise NotImplementedError("write your pallas kernel here")


# speedup vs baseline: 5.9343x; 5.9343x over previous
import functools

import jax
import jax.numpy as jnp
from jax import lax
from jax.experimental import pallas as pl
from jax.experimental.pallas import tpu as pltpu
from jax.experimental.pallas import tpu_sc as plsc

N = 10000
E = 320000
D = 128
H = 128
OUT = 10
G = 64
BN_EPS = 1e-5

CHUNK = 80
BLK = 1000
NBLK = N // BLK


ZROWS = 80


def _sc_body(nc, ns, x_hbm, src_hbm, dst_hbm, zeros_hbm, out_hbm,
             sidx_v, didx_v, rows_v, acc_sh, sem):
    c = lax.axis_index("c")
    s = lax.axis_index("s")
    wid = s * nc + c
    nw = nc * ns
    chunks_per_tile = E // (CHUNK * nw)
    nzch = N // ZROWS

    for j in range((nzch + ns - 1) // ns):
        k = j * ns + s

        @pl.when(k < nzch)
        def _():
            pltpu.sync_copy(zeros_hbm, acc_sh.at[pl.ds(k * ZROWS, ZROWS)])

    pltpu.sync_copy(src_hbm.at[wid], sidx_v)
    pltpu.sync_copy(dst_hbm.at[wid], didx_v)
    plsc.subcore_barrier()

    def body(i, carry):
        pltpu.async_copy(x_hbm.at[sidx_v.at[i]], rows_v, sem).wait()
        pltpu.sync_copy(rows_v, acc_sh.at[didx_v.at[i]], add=True)
        return carry

    lax.fori_loop(0, chunks_per_tile, body, 0)
    plsc.subcore_barrier()
    for j in range((nzch + ns - 1) // ns):
        k = j * ns + s

        @pl.when(k < nzch)
        def _():
            pltpu.sync_copy(acc_sh.at[pl.ds(k * ZROWS, ZROWS)],
                            out_hbm.at[c, pl.ds(k * ZROWS, ZROWS)])


def _sc_segment_sum(x, src3, dst3, zeros):
    try:
        info = plsc.get_sparse_core_info()
        nc, ns = info.num_cores, info.num_subcores
    except Exception:
        nc, ns = 2, 16
    mesh = plsc.VectorSubcoreMesh(core_axis_name="c", subcore_axis_name="s")
    nw = nc * ns
    chunks_per_tile = E // (CHUNK * nw)
    rows_per_sub = N // ns
    kern = pl.kernel(
        functools.partial(_sc_body, nc, ns),
        out_type=jax.ShapeDtypeStruct((nc, N, D), jnp.float32),
        mesh=mesh,
        scratch_types=[
            pltpu.VMEM((chunks_per_tile, CHUNK), jnp.int32),
            pltpu.VMEM((chunks_per_tile, CHUNK), jnp.int32),
            pltpu.VMEM((CHUNK, D), jnp.float32),
            pltpu.VMEM_SHARED((N, D), jnp.float32),
            pltpu.SemaphoreType.DMA,
        ],
    )
    return kern(x, src3, dst3, zeros)


def _tc_layer_body(x_ref, a0_ref, a1_ref, w1_ref, b1_ref, g_ref, be_ref,
                   w2_ref, b2_ref, batch_ref, xout_ref, pooled_ref,
                   h1_scr, sums_scr, bn_scr):
    p = pl.program_id(0)
    i = pl.program_id(1)

    @pl.when(p == 0)
    def _phase0():
        h = x_ref[...] + a0_ref[...] + a1_ref[...]
        h1 = lax.dot_general(h.astype(jnp.bfloat16),
                             w1_ref[...].astype(jnp.bfloat16),
                             (((1,), (0,)), ((), ())),
                             preferred_element_type=jnp.float32) + b1_ref[...]
        h1_scr[i] = h1
        s1 = jnp.sum(h1, axis=0, keepdims=True)
        s2 = jnp.sum(h1 * h1, axis=0, keepdims=True)
        blk_sums = jnp.concatenate([s1, s2], axis=0)

        @pl.when(i == 0)
        def _():
            sums_scr[...] = blk_sums

        @pl.when(i != 0)
        def _():
            sums_scr[...] += blk_sums

    @pl.when(p == 1)
    def _phase1():
        @pl.when(i == 0)
        def _():
            mean = sums_scr[0:1, :] / float(N)
            var = sums_scr[1:2, :] / float(N) - mean * mean
            inv = lax.rsqrt(var + BN_EPS)
            scale = g_ref[...] * inv
            shift = be_ref[...] - mean * scale
            bn_scr[...] = jnp.concatenate([scale, shift], axis=0)

        hn = jax.nn.relu(h1_scr[i] * bn_scr[0:1, :] + bn_scr[1:2, :])
        h2 = lax.dot_general(hn.astype(jnp.bfloat16),
                             w2_ref[...].astype(jnp.bfloat16),
                             (((1,), (0,)), ((), ())),
                             preferred_element_type=jnp.float32) + b2_ref[...]
        xo = jax.nn.relu(h2)
        xout_ref[...] = xo
        gids = lax.broadcasted_iota(jnp.int32, (G, BLK), 0)
        onehot = (gids == batch_ref[0]).astype(jnp.float32)
        pb = lax.dot_general(onehot, xo, (((1,), (0,)), ((), ())),
                             precision=lax.Precision.HIGHEST,
                             preferred_element_type=jnp.float32)

        @pl.when(i == 0)
        def _():
            pooled_ref[...] = pb

        @pl.when(i != 0)
        def _():
            pooled_ref[...] += pb


def _tc_layer(x, agg0, agg1, batch3, p):
    row = lambda pi, i: (i * pi, 0)
    full = lambda pi, i: (0, 0)
    return pl.pallas_call(
        _tc_layer_body,
        grid=(2, NBLK),
        in_specs=[
            pl.BlockSpec((BLK, D), lambda pi, i: (i, 0)),
            pl.BlockSpec((BLK, D), lambda pi, i: (i, 0)),
            pl.BlockSpec((BLK, D), lambda pi, i: (i, 0)),
            pl.BlockSpec((D, H), full),
            pl.BlockSpec((1, H), full),
            pl.BlockSpec((1, H), full),
            pl.BlockSpec((1, H), full),
            pl.BlockSpec((H, H), full),
            pl.BlockSpec((1, H), full),
            pl.BlockSpec((1, 1, BLK), lambda pi, i: (i, 0, 0)),
        ],
        out_specs=[
            pl.BlockSpec((BLK, H), row),
            pl.BlockSpec((G, H), full),
        ],
        out_shape=[
            jax.ShapeDtypeStruct((N, H), jnp.float32),
            jax.ShapeDtypeStruct((G, H), jnp.float32),
        ],
        scratch_shapes=[
            pltpu.VMEM((NBLK, BLK, H), jnp.float32),
            pltpu.VMEM((2, H), jnp.float32),
            pltpu.VMEM((2, H), jnp.float32),
        ],
    )(x, agg0, agg1, p["W1"], p["b1"].reshape(1, H), p["g"].reshape(1, H),
      p["be"].reshape(1, H), p["W2"], p["b2"].reshape(1, H), batch3)


def _head_body(p1_ref, p2_ref, p3_ref, wc_ref, bc_ref, ww1_ref, bw1_ref,
               ww2_ref, bw2_ref, cls_ref, wm_ref):
    pooled = jnp.concatenate([p1_ref[...], p2_ref[...], p3_ref[...]], axis=1)
    mm = lambda a, b: lax.dot_general(a.astype(jnp.bfloat16),
                                      b.astype(jnp.bfloat16),
                                      (((1,), (0,)), ((), ())),
                                      preferred_element_type=jnp.float32)
    cls_ref[...] = mm(pooled, wc_ref[...]) + bc_ref[...]
    wmh = jax.nn.relu(mm(pooled, ww1_ref[...]) + bw1_ref[...])
    z = mm(wmh, ww2_ref[...]) + bw2_ref[...]
    wm_ref[...] = 1.0 / (1.0 + jnp.exp(-z))


def _head(p1, p2, p3, p):
    return pl.pallas_call(
        _head_body,
        out_shape=[
            jax.ShapeDtypeStruct((G, OUT), jnp.float32),
            jax.ShapeDtypeStruct((G, 1), jnp.float32),
        ],
    )(p1, p2, p3, p["Wc"], p["bc"].reshape(1, OUT),
      p["Ww1"], p["bw1"].reshape(1, H // 2), p["Ww2"],
      p["bw2"].reshape(1, 1))


def kernel(x, edge_index, batch, params):
    nw = 32
    chunks_per_tile = E // (CHUNK * nw)
    src3 = edge_index[0].reshape(nw, chunks_per_tile, CHUNK)
    dst3 = edge_index[1].reshape(nw, chunks_per_tile, CHUNK)
    zeros = jnp.zeros((ZROWS, D), jnp.float32)
    batch3 = batch.reshape(NBLK, 1, BLK)

    xl = x
    pooled = []
    for name in ("c1", "c2", "c3"):
        parts = _sc_segment_sum(xl, src3, dst3, zeros)
        xl, pg = _tc_layer(xl, parts[0], parts[1], batch3, params[name])
        pooled.append(pg)

    return _head(pooled[0], pooled[1], pooled[2], params)
